# baseline (speedup 1.0000x reference)
import functools

import jax
import jax.numpy as jnp
from jax import lax
from jax.experimental import pallas as pl
from jax.experimental.pallas import tpu as pltpu
from jax.experimental.pallas import tpu_sc as plsc

NC = 2
NS = 16
NW = NC * NS
K = 128


def _deg_dinv_kernel(n_pad, n_slab):
    rows = n_pad // NS
    n_vec = n_slab // 16
    mesh = plsc.VectorSubcoreMesh(core_axis_name="c", subcore_axis_name="s")

    @functools.partial(
        pl.kernel,
        out_type=jax.ShapeDtypeStruct((NS, rows), jnp.float32),
        mesh=mesh,
        scratch_types=[
            pltpu.VMEM((n_slab,), jnp.int32),
            pltpu.VMEM((n_pad,), jnp.float32),
            pltpu.VMEM((NS, rows), jnp.float32),
            pltpu.VMEM((rows,), jnp.float32),
            pltpu.VMEM_SHARED((NS, n_pad), jnp.float32),
        ],
        compiler_params=pltpu.CompilerParams(needs_layout_passes=False),
    )
    def deg_kernel(dst_hbm, out_hbm, idx_v, deg_v, buf_v, dinv_v, hist_sh):
        cid = lax.axis_index("c")
        sid = lax.axis_index("s")
        pltpu.sync_copy(dst_hbm.at[sid], idx_v)
        zeros16 = jnp.zeros((16,), jnp.float32)
        ones16 = jnp.ones((16,), jnp.float32)

        def zbody(i, carry):
            deg_v[pl.ds(i * 16, 16)] = zeros16
            return carry

        lax.fori_loop(0, n_pad // 16, zbody, 0)

        def body(t, carry):
            iv = idx_v[pl.ds(t * 16, 16)]
            plsc.addupdate_scatter(deg_v, [iv], ones16)
            return carry

        lax.fori_loop(0, n_vec, body, 0)
        pltpu.sync_copy(deg_v, hist_sh.at[sid])
        plsc.subcore_barrier()
        pltpu.sync_copy(hist_sh.at[:, pl.ds(sid * rows, rows)], buf_v)

        def rbody(i, carry):
            sl16 = pl.ds(i * 16, 16)
            dg = ones16
            for r in range(NS):
                dg = dg + buf_v[r, sl16]
            yi = jnp.int32(0x5F3759DF) - lax.shift_right_logical(
                lax.bitcast_convert_type(dg, jnp.int32), 1)
            y = lax.bitcast_convert_type(yi, jnp.float32)
            for _ in range(3):
                y = y * (1.5 - 0.5 * dg * y * y)
            dinv_v[sl16] = y
            return carry

        lax.fori_loop(0, rows // 16, rbody, 0)

        @pl.when(cid == 0)
        def _():
            pltpu.sync_copy(dinv_v, out_hbm.at[sid])

    return deg_kernel


def _agg_kernel(n_pad, n_chunks, d, tc_tiling=True):
    rows = n_pad // NS
    mesh = plsc.VectorSubcoreMesh(core_axis_name="c", subcore_axis_name="s")

    nh = n_chunks // 2

    @functools.partial(
        pl.kernel,
        out_type=jax.ShapeDtypeStruct((NC, n_pad, d), jnp.float32),
        mesh=mesh,
        scratch_types=[
            pltpu.VMEM((nh, K), jnp.int32),
            pltpu.VMEM((nh, K), jnp.int32),
            pltpu.VMEM((K, d), jnp.float32),
            pltpu.VMEM((K, d), jnp.float32),
            pltpu.VMEM_SHARED((n_pad, d), jnp.float32),
            pltpu.SemaphoreType.DMA,
            pltpu.SemaphoreType.DMA,
            pltpu.SemaphoreType.DMA,
            pltpu.SemaphoreType.DMA,
        ],
        compiler_params=pltpu.CompilerParams(use_tc_tiling_on_sc=tc_tiling),
    )
    def agg_kernel(src_hbm, dst_hbm, z_hbm, zeros_hbm, out_hbm,
                   src_v, dst_v, r0, r1, acc_sh, g0, g1, s0, s1):
        cid = lax.axis_index("c")
        sid = lax.axis_index("s")
        wid = cid * NS + sid
        sl = pl.ds(sid * rows, rows)
        pltpu.sync_copy(zeros_hbm.at[sl], acc_sh.at[sl])
        plsc.subcore_barrier()

        for half in range(2):
            pltpu.sync_copy(src_hbm.at[wid, pl.ds(half * nh, nh)], src_v)
            pltpu.sync_copy(dst_hbm.at[wid, pl.ds(half * nh, nh)], dst_v)
            pltpu.async_copy(z_hbm.at[src_v.at[0]], r0, g0).wait()

            def body(t, carry):
                j0 = 2 * t
                j1 = j0 + 1
                j2 = lax.rem(j0 + 2, nh)
                sd0 = pltpu.async_copy(r0, acc_sh.at[dst_v.at[j0]], s0,
                                       add=True)
                gd1 = pltpu.async_copy(z_hbm.at[src_v.at[j1]], r1, g1)
                gd1.wait()
                sd0.wait()
                sd1 = pltpu.async_copy(r1, acc_sh.at[dst_v.at[j1]], s1,
                                       add=True)
                gd2 = pltpu.async_copy(z_hbm.at[src_v.at[j2]], r0, g0)
                gd2.wait()
                sd1.wait()
                return carry

            lax.fori_loop(0, nh // 2, body, 0)
        plsc.subcore_barrier()
        pltpu.sync_copy(acc_sh.at[sl], out_hbm.at[cid, sl])

    return agg_kernel


def _agg_kernel_big(n_pad, n_chunks, d):
    rows = n_pad // NS
    bc = 4 * K
    nb = n_chunks // 4
    mesh = plsc.VectorSubcoreMesh(core_axis_name="c", subcore_axis_name="s")

    @functools.partial(
        pl.kernel,
        out_type=jax.ShapeDtypeStruct((NC, n_pad, d), jnp.float32),
        mesh=mesh,
        scratch_types=[
            pltpu.VMEM((n_chunks * K,), jnp.int32),
            pltpu.VMEM((n_chunks, K), jnp.int32),
            pltpu.VMEM((bc, d), jnp.float32),
            pltpu.VMEM((bc, d), jnp.float32),
            pltpu.VMEM_SHARED((n_pad, d), jnp.float32),
            pltpu.SemaphoreType.DMA,
            pltpu.SemaphoreType.DMA,
            pltpu.SemaphoreType.DMA,
            pltpu.SemaphoreType.DMA,
        ],
        compiler_params=pltpu.CompilerParams(use_tc_tiling_on_sc=False),
    )
    def agg_kernel(srcf_hbm, dst_hbm, z_hbm, zeros_hbm, out_hbm,
                   src_v, dst_v, r0, r1, acc_sh, g0, g1, s0, s1):
        cid = lax.axis_index("c")
        sid = lax.axis_index("s")
        wid = cid * NS + sid
        pltpu.sync_copy(srcf_hbm.at[wid], src_v)
        pltpu.sync_copy(dst_hbm.at[wid], dst_v)
        sl = pl.ds(sid * rows, rows)
        pltpu.sync_copy(zeros_hbm.at[sl], acc_sh.at[sl])
        plsc.subcore_barrier()

        def gather(j, r, sem):
            return pltpu.async_copy(
                z_hbm.at[src_v.at[pl.ds(j * bc, bc)]], r, sem)

        def scatter4(r, j, sem):
            return [
                pltpu.async_copy(r.at[pl.ds(q * K, K)],
                                 acc_sh.at[dst_v.at[4 * j + q]], sem,
                                 add=True)
                for q in range(4)
            ]

        gather(0, r0, g0).wait()

        def body(t, carry):
            j0 = 2 * t
            j1 = j0 + 1
            j2 = lax.rem(j0 + 2, nb)
            sd0 = scatter4(r0, j0, s0)
            gd1 = gather(j1, r1, g1)
            gd1.wait()
            for sd in sd0:
                sd.wait()
            sd1 = scatter4(r1, j1, s1)
            gd2 = gather(j2, r0, g0)
            gd2.wait()
            for sd in sd1:
                sd.wait()
            return carry

        lax.fori_loop(0, nb // 2, body, 0)
        plsc.subcore_barrier()
        pltpu.sync_copy(acc_sh.at[sl], out_hbm.at[cid, sl])

    return agg_kernel


def _zw_body(x_ref, w_ref, dinv_ref, out_ref):
    xw = jnp.dot(x_ref[...], w_ref[...], preferred_element_type=jnp.float32)
    out_ref[...] = xw * dinv_ref[...]


def _mid_body(agg_ref, z1_ref, dinv_ref, b1_ref, w2_ref, out_ref):
    s = agg_ref[0] + agg_ref[1] + z1_ref[...]
    h = jnp.maximum(s * dinv_ref[...] + b1_ref[...], 0.0)
    hw = jnp.dot(h, w2_ref[...], preferred_element_type=jnp.float32)
    out_ref[...] = hw * dinv_ref[...]


def _final_body(agg_ref, z2_ref, dinv_ref, b2_ref, out_ref):
    s = agg_ref[0] + agg_ref[1] + z2_ref[...]
    out_ref[...] = s * dinv_ref[...] + b2_ref[...]


def kernel(x, edge_index, W1, b1, W2, b2):
    n, d_in = x.shape
    d_h = W1.shape[1]
    d_out = W2.shape[1]
    e = edge_index.shape[1]

    n_pad = ((n + 255) // 256) * 256
    n_extra = n_pad - n
    n_chunks = -(-e // (NW * K))
    n_chunks = -(-n_chunks // 8) * 8
    e_pad = NW * K * n_chunks

    ei = edge_index.astype(jnp.int32)
    pad = jnp.full((e_pad - e,), n, jnp.int32) + (
        jnp.arange(e_pad - e, dtype=jnp.int32) % jnp.int32(max(n_extra, 1)))
    src = jnp.concatenate([ei[0], pad]).reshape(NW, n_chunks, K)
    dst = jnp.concatenate([ei[1], pad]).reshape(NW, n_chunks, K)

    x_pad = jnp.concatenate([x, jnp.zeros((n_extra, d_in), x.dtype)])
    zeros_h = jnp.zeros((n_pad, d_h), jnp.float32)

    dinv = _deg_dinv_kernel(n_pad, e_pad // NS)(dst.reshape(NS, e_pad // NS))
    dinv_col = dinv.reshape(n_pad, 1)

    br = n_pad // 8
    grid = (n_pad // br,)

    z1 = pl.pallas_call(
        _zw_body,
        grid=grid,
        in_specs=[
            pl.BlockSpec((br, d_in), lambda i: (i, 0)),
            pl.BlockSpec((d_in, d_h), lambda i: (0, 0)),
            pl.BlockSpec((br, 1), lambda i: (i, 0)),
        ],
        out_specs=pl.BlockSpec((br, d_h), lambda i: (i, 0)),
        out_shape=jax.ShapeDtypeStruct((n_pad, d_h), jnp.float32),
    )(x_pad, W1, dinv_col)

    agg1 = _agg_kernel(n_pad, n_chunks, d_h)(src, dst, z1, zeros_h)

    z2 = pl.pallas_call(
        _mid_body,
        grid=grid,
        in_specs=[
            pl.BlockSpec((NC, br, d_h), lambda i: (0, i, 0)),
            pl.BlockSpec((br, d_h), lambda i: (i, 0)),
            pl.BlockSpec((br, 1), lambda i: (i, 0)),
            pl.BlockSpec((1, d_h), lambda i: (0, 0)),
            pl.BlockSpec((d_h, d_out), lambda i: (0, 0)),
        ],
        out_specs=pl.BlockSpec((br, d_out), lambda i: (i, 0)),
        out_shape=jax.ShapeDtypeStruct((n_pad, d_out), jnp.float32),
    )(agg1, z1, dinv_col, b1.reshape(1, d_h), W2)

    zeros_o = jnp.zeros((n_pad, d_out), jnp.float32)
    agg2 = _agg_kernel_big(n_pad, n_chunks, d_out)(
        src.reshape(NW, n_chunks * K), dst, z2, zeros_o)

    out = pl.pallas_call(
        _final_body,
        grid=grid,
        in_specs=[
            pl.BlockSpec((NC, br, d_out), lambda i: (0, i, 0)),
            pl.BlockSpec((br, d_out), lambda i: (i, 0)),
            pl.BlockSpec((br, 1), lambda i: (i, 0)),
            pl.BlockSpec((1, d_out), lambda i: (0, 0)),
        ],
        out_specs=pl.BlockSpec((br, d_out), lambda i: (i, 0)),
        out_shape=jax.ShapeDtypeStruct((n_pad, d_out), jnp.float32),
    )(agg2, z2, dinv_col, b2.reshape(1, d_out))

    return out[:n]


# speedup vs baseline: 1.0265x; 1.0265x over previous
import functools

import jax
import jax.numpy as jnp
from jax import lax
from jax.experimental import pallas as pl
from jax.experimental.pallas import tpu as pltpu
from jax.experimental.pallas import tpu_sc as plsc

NC = 2
NS = 16
NW = NC * NS
K = 128


def _degree_kernel(n_pad, n_chunks):
    n_vec = n_chunks * K // 16
    mesh = plsc.VectorSubcoreMesh(core_axis_name="c", subcore_axis_name="s")

    @functools.partial(
        pl.kernel,
        out_type=jax.ShapeDtypeStruct((NW, n_pad), jnp.float32),
        mesh=mesh,
        scratch_types=[
            pltpu.VMEM((n_chunks * K,), jnp.int32),
            pltpu.VMEM((n_pad,), jnp.float32),
        ],
        compiler_params=pltpu.CompilerParams(needs_layout_passes=False),
    )
    def deg_kernel(dst_hbm, out_hbm, idx_v, deg_v):
        cid = lax.axis_index("c")
        sid = lax.axis_index("s")
        wid = cid * NS + sid
        pltpu.sync_copy(dst_hbm.at[wid], idx_v)
        zeros16 = jnp.zeros((16,), jnp.float32)
        ones16 = jnp.ones((16,), jnp.float32)

        def zbody(i, carry):
            deg_v[pl.ds(i * 16, 16)] = zeros16
            return carry

        lax.fori_loop(0, n_pad // 16, zbody, 0)

        def body(t, carry):
            iv = idx_v[pl.ds(t * 16, 16)]
            plsc.addupdate_scatter(deg_v, [iv], ones16)
            return carry

        lax.fori_loop(0, n_vec, body, 0)
        pltpu.sync_copy(deg_v, out_hbm.at[wid])

    return deg_kernel


def _agg_kernel(n_pad, n_chunks, d, tc_tiling=True):
    rows = n_pad // NS
    mesh = plsc.VectorSubcoreMesh(core_axis_name="c", subcore_axis_name="s")

    nh = n_chunks // 2

    @functools.partial(
        pl.kernel,
        out_type=jax.ShapeDtypeStruct((NC, n_pad, d), jnp.float32),
        mesh=mesh,
        scratch_types=[
            pltpu.VMEM((nh, K), jnp.int32),
            pltpu.VMEM((nh, K), jnp.int32),
            pltpu.VMEM((K, d), jnp.float32),
            pltpu.VMEM((K, d), jnp.float32),
            pltpu.VMEM_SHARED((n_pad, d), jnp.float32),
            pltpu.SemaphoreType.DMA,
            pltpu.SemaphoreType.DMA,
            pltpu.SemaphoreType.DMA,
            pltpu.SemaphoreType.DMA,
        ],
        compiler_params=pltpu.CompilerParams(use_tc_tiling_on_sc=tc_tiling),
    )
    def agg_kernel(src_hbm, dst_hbm, z_hbm, zeros_hbm, out_hbm,
                   src_v, dst_v, r0, r1, acc_sh, g0, g1, s0, s1):
        cid = lax.axis_index("c")
        sid = lax.axis_index("s")
        wid = cid * NS + sid
        sl = pl.ds(sid * rows, rows)
        pltpu.sync_copy(zeros_hbm.at[sl], acc_sh.at[sl])
        plsc.subcore_barrier()

        for half in range(2):
            pltpu.sync_copy(src_hbm.at[wid, pl.ds(half * nh, nh)], src_v)
            pltpu.sync_copy(dst_hbm.at[wid, pl.ds(half * nh, nh)], dst_v)
            pltpu.async_copy(z_hbm.at[src_v.at[0]], r0, g0).wait()

            def body(t, carry):
                j0 = 2 * t
                j1 = j0 + 1
                j2 = lax.rem(j0 + 2, nh)
                sd0 = pltpu.async_copy(r0, acc_sh.at[dst_v.at[j0]], s0,
                                       add=True)
                gd1 = pltpu.async_copy(z_hbm.at[src_v.at[j1]], r1, g1)
                gd1.wait()
                sd0.wait()
                sd1 = pltpu.async_copy(r1, acc_sh.at[dst_v.at[j1]], s1,
                                       add=True)
                gd2 = pltpu.async_copy(z_hbm.at[src_v.at[j2]], r0, g0)
                gd2.wait()
                sd1.wait()
                return carry

            lax.fori_loop(0, nh // 2, body, 0)
        plsc.subcore_barrier()
        pltpu.sync_copy(acc_sh.at[sl], out_hbm.at[cid, sl])

    return agg_kernel


def _agg_kernel_big(n_pad, n_chunks, d):
    rows = n_pad // NS
    bc = 4 * K
    nb = n_chunks // 4
    mesh = plsc.VectorSubcoreMesh(core_axis_name="c", subcore_axis_name="s")

    @functools.partial(
        pl.kernel,
        out_type=jax.ShapeDtypeStruct((NC, n_pad, d), jnp.float32),
        mesh=mesh,
        scratch_types=[
            pltpu.VMEM((n_chunks * K,), jnp.int32),
            pltpu.VMEM((n_chunks, K), jnp.int32),
            pltpu.VMEM((bc, d), jnp.float32),
            pltpu.VMEM((bc, d), jnp.float32),
            pltpu.VMEM_SHARED((n_pad, d), jnp.float32),
            pltpu.SemaphoreType.DMA,
            pltpu.SemaphoreType.DMA,
            pltpu.SemaphoreType.DMA,
            pltpu.SemaphoreType.DMA,
        ],
        compiler_params=pltpu.CompilerParams(use_tc_tiling_on_sc=False),
    )
    def agg_kernel(srcf_hbm, dst_hbm, z_hbm, zeros_hbm, out_hbm,
                   src_v, dst_v, r0, r1, acc_sh, g0, g1, s0, s1):
        cid = lax.axis_index("c")
        sid = lax.axis_index("s")
        wid = cid * NS + sid
        pltpu.sync_copy(srcf_hbm.at[wid], src_v)
        pltpu.sync_copy(dst_hbm.at[wid], dst_v)
        sl = pl.ds(sid * rows, rows)
        pltpu.sync_copy(zeros_hbm.at[sl], acc_sh.at[sl])
        plsc.subcore_barrier()

        def gather(j, r, sem):
            return pltpu.async_copy(
                z_hbm.at[src_v.at[pl.ds(j * bc, bc)]], r, sem)

        def scatter4(r, j, sem):
            return [
                pltpu.async_copy(r.at[pl.ds(q * K, K)],
                                 acc_sh.at[dst_v.at[4 * j + q]], sem,
                                 add=True)
                for q in range(4)
            ]

        gather(0, r0, g0).wait()

        def body(t, carry):
            j0 = 2 * t
            j1 = j0 + 1
            j2 = lax.rem(j0 + 2, nb)
            sd0 = scatter4(r0, j0, s0)
            gd1 = gather(j1, r1, g1)
            gd1.wait()
            for sd in sd0:
                sd.wait()
            sd1 = scatter4(r1, j1, s1)
            gd2 = gather(j2, r0, g0)
            gd2.wait()
            for sd in sd1:
                sd.wait()
            return carry

        lax.fori_loop(0, nb // 2, body, 0)
        plsc.subcore_barrier()
        pltpu.sync_copy(acc_sh.at[sl], out_hbm.at[cid, sl])

    return agg_kernel


def _dinv_body(p_ref, out_ref):
    deg = jnp.sum(p_ref[...], axis=0, keepdims=True) + 1.0
    out_ref[...] = lax.rsqrt(deg)


def _zw_body(x_ref, w_ref, dinv_ref, out_ref):
    xw = jnp.dot(x_ref[...], w_ref[...], preferred_element_type=jnp.float32)
    out_ref[...] = xw * dinv_ref[...]


def _mid_body(agg_ref, z1_ref, dinv_ref, b1_ref, w2_ref, out_ref):
    s = agg_ref[0] + agg_ref[1] + z1_ref[...]
    h = jnp.maximum(s * dinv_ref[...] + b1_ref[...], 0.0)
    hw = jnp.dot(h, w2_ref[...], preferred_element_type=jnp.float32)
    out_ref[...] = hw * dinv_ref[...]


def _final_body(agg_ref, z2_ref, dinv_ref, b2_ref, out_ref):
    s = agg_ref[0] + agg_ref[1] + z2_ref[...]
    out_ref[...] = s * dinv_ref[...] + b2_ref[...]


def kernel(x, edge_index, W1, b1, W2, b2):
    n, d_in = x.shape
    d_h = W1.shape[1]
    d_out = W2.shape[1]
    e = edge_index.shape[1]

    n_pad = ((n + 127) // 128) * 128
    n_extra = n_pad - n
    n_chunks = -(-e // (NW * K))
    n_chunks = -(-n_chunks // 8) * 8
    e_pad = NW * K * n_chunks

    ei = edge_index.astype(jnp.int32)
    pad = jnp.full((e_pad - e,), n, jnp.int32) + (
        jnp.arange(e_pad - e, dtype=jnp.int32) % jnp.int32(max(n_extra, 1)))
    src = jnp.concatenate([ei[0], pad]).reshape(NW, n_chunks, K)
    dst = jnp.concatenate([ei[1], pad]).reshape(NW, n_chunks, K)

    x_pad = jnp.concatenate([x, jnp.zeros((n_extra, d_in), x.dtype)])
    zeros_h = jnp.zeros((n_pad, d_h), jnp.float32)

    degp = _degree_kernel(n_pad, n_chunks)(dst.reshape(NW, n_chunks * K))

    dinv_row = pl.pallas_call(
        _dinv_body,
        out_shape=jax.ShapeDtypeStruct((1, n_pad), jnp.float32),
    )(degp)
    dinv_col = dinv_row.reshape(n_pad, 1)

    br = n_pad // 8
    grid = (n_pad // br,)

    z1 = pl.pallas_call(
        _zw_body,
        grid=grid,
        in_specs=[
            pl.BlockSpec((br, d_in), lambda i: (i, 0)),
            pl.BlockSpec((d_in, d_h), lambda i: (0, 0)),
            pl.BlockSpec((br, 1), lambda i: (i, 0)),
        ],
        out_specs=pl.BlockSpec((br, d_h), lambda i: (i, 0)),
        out_shape=jax.ShapeDtypeStruct((n_pad, d_h), jnp.float32),
    )(x_pad, W1, dinv_col)

    agg1 = _agg_kernel(n_pad, n_chunks, d_h)(src, dst, z1, zeros_h)

    z2 = pl.pallas_call(
        _mid_body,
        grid=grid,
        in_specs=[
            pl.BlockSpec((NC, br, d_h), lambda i: (0, i, 0)),
            pl.BlockSpec((br, d_h), lambda i: (i, 0)),
            pl.BlockSpec((br, 1), lambda i: (i, 0)),
            pl.BlockSpec((1, d_h), lambda i: (0, 0)),
            pl.BlockSpec((d_h, d_out), lambda i: (0, 0)),
        ],
        out_specs=pl.BlockSpec((br, d_out), lambda i: (i, 0)),
        out_shape=jax.ShapeDtypeStruct((n_pad, d_out), jnp.float32),
    )(agg1, z1, dinv_col, b1.reshape(1, d_h), W2)

    zeros_o = jnp.zeros((n_pad, d_out), jnp.float32)
    agg2 = _agg_kernel_big(n_pad, n_chunks, d_out)(
        src.reshape(NW, n_chunks * K), dst, z2, zeros_o)

    out = pl.pallas_call(
        _final_body,
        grid=grid,
        in_specs=[
            pl.BlockSpec((NC, br, d_out), lambda i: (0, i, 0)),
            pl.BlockSpec((br, d_out), lambda i: (i, 0)),
            pl.BlockSpec((br, 1), lambda i: (i, 0)),
            pl.BlockSpec((1, d_out), lambda i: (0, 0)),
        ],
        out_specs=pl.BlockSpec((br, d_out), lambda i: (i, 0)),
        out_shape=jax.ShapeDtypeStruct((n_pad, d_out), jnp.float32),
    )(agg2, z2, dinv_col, b2.reshape(1, d_out))

    return out[:n]
